# baseline (speedup 1.0000x reference)
import functools

import jax
import jax.numpy as jnp
from jax import lax
from jax.experimental import pallas as pl
from jax.experimental.pallas import tpu as pltpu
from jax.experimental.pallas import tpu_sc as plsc

N = 100000
D = 128
B = 512
NC = 2
NS = 16
NW = NC * NS
L = 16
ND = D // L

CH = 448
T = -(-N // CH)
TAIL = N - (T - 1) * CH
JMAX = -(-T // NW)
W = 2 * D + L
ZR = B // NS
AC = 2 * D

_MESH = plsc.VectorSubcoreMesh(
    core_axis_name="c", subcore_axis_name="s", num_cores=NC, num_subcores=NS)


def _rsqrt_newton(v):
    i = plsc.bitcast(v, jnp.int32)
    i = jnp.int32(0x5F3759DF) - lax.shift_right_logical(i, 1)
    y = plsc.bitcast(i, jnp.float32)
    for _ in range(3):
        y = y * (1.5 - 0.5 * v * y * y)
    return y


def _stats_body(feat_hbm, seg_hbm, zero_hbm, part_hbm,
                featbuf, segbuf, stage, acc_sh):
    cid = lax.axis_index("c")
    sid = lax.axis_index("s")
    wid = sid * NC + cid

    pltpu.sync_copy(zero_hbm, acc_sh.at[pl.ds(sid * ZR, ZR)])
    pltpu.sync_copy(zero_hbm.at[pl.ds(0, L)], stage)
    plsc.subcore_barrier()

    lane = lax.iota(jnp.int32, L)
    row0 = jnp.zeros((L,), jnp.int32)

    def _flush(cur, cnt, accs, sqs):
        for j in range(ND):
            plsc.store_scatter(stage, [row0, lane + j * L], accs[j])
            plsc.store_scatter(stage, [row0, lane + D + j * L], sqs[j])
        plsc.store_scatter(stage, [row0, lane + 2 * D],
                           jnp.full((L,), cnt, jnp.float32))
        idxvec = jnp.full((L,), cur, jnp.int32)
        pltpu.sync_copy(stage, acc_sh.at[idxvec], add=True)

    def _proc(c, rows):
        base = c * CH
        pltpu.sync_copy(feat_hbm.at[pl.ds(base * D, rows * D)],
                        featbuf.at[pl.ds(0, rows * D)])
        pltpu.sync_copy(seg_hbm.at[pl.ds(base, rows)],
                        segbuf.at[pl.ds(0, rows)])

        zv = jnp.zeros((L,), jnp.float32)
        carry0 = ((segbuf[pl.ds(0, L)][0], jnp.float32(0.0))
                  + (zv,) * (2 * ND))

        def _group(g, carry):
            cur, cnt = carry[0], carry[1]
            accs = list(carry[2:2 + ND])
            sqs = list(carry[2 + ND:])
            segs = segbuf[pl.ds(g * L, L)]
            for k in range(L):
                s_k = segs[k]
                prev = cur if k == 0 else segs[k - 1]
                is_new = s_k != prev

                @pl.when(is_new)
                def _(cnt=cnt, accs=tuple(accs), sqs=tuple(sqs), prev=prev):
                    _flush(prev, cnt, accs, sqs)

                r = g * L + k
                xs = [featbuf[pl.ds(r * D + j * L, L)] for j in range(ND)]
                cnt = jnp.where(is_new, 1.0, cnt + 1.0)
                for j in range(ND):
                    accs[j] = jnp.where(is_new, xs[j], accs[j] + xs[j])
                    sqs[j] = jnp.where(is_new, xs[j] * xs[j],
                                       sqs[j] + xs[j] * xs[j])
            return (segs[L - 1], cnt) + tuple(accs) + tuple(sqs)

        end = lax.fori_loop(0, rows // L, _group, carry0)
        _flush(end[0], end[1], end[2:2 + ND], end[2 + ND:])

    def _trip(j, _):
        c = wid + j * NW

        @pl.when(c < T - 1)
        def _():
            _proc(c, CH)

        @pl.when(c == T - 1)
        def _():
            _proc(c, TAIL)
        return 0
    lax.fori_loop(0, JMAX, _trip, 0)

    plsc.subcore_barrier()
    pltpu.sync_copy(acc_sh.at[pl.ds(sid * ZR, ZR)],
                    part_hbm.at[pl.ds(cid * B + sid * ZR, ZR)])


def _apply_body(feat_hbm, seg_hbm, part_hbm, w_hbm, b_hbm, ms_hbm, out_hbm,
                buf, segbuf, acrow, p0buf, p1buf, acstage,
                wbuf, bbuf, msbuf, ac_sh):
    cid = lax.axis_index("c")
    sid = lax.axis_index("s")
    wid = sid * NC + cid

    pltpu.sync_copy(w_hbm, wbuf)
    pltpu.sync_copy(b_hbm, bbuf)
    pltpu.sync_copy(ms_hbm, msbuf)
    pltpu.sync_copy(part_hbm.at[pl.ds(sid * ZR, ZR)], p0buf)
    pltpu.sync_copy(part_hbm.at[pl.ds(B + sid * ZR, ZR)], p1buf)

    def _seg(s, _):
        cntv = p0buf[s, pl.ds(2 * D, L)] + p1buf[s, pl.ds(2 * D, L)]
        cnt = jnp.maximum(cntv, 1.0)
        for j in range(ND):
            sm = p0buf[s, pl.ds(j * L, L)] + p1buf[s, pl.ds(j * L, L)]
            sq = (p0buf[s, pl.ds(D + j * L, L)]
                  + p1buf[s, pl.ds(D + j * L, L)])
            mean = sm / cnt
            m = mean * msbuf[pl.ds(j * L, L)]
            var = sq / cnt - m * (2.0 * mean - m)
            y = _rsqrt_newton(var + 1e-6)
            a = wbuf[pl.ds(j * L, L)] * y
            acstage[pl.ds(s * AC + j * L, L)] = a
            acstage[pl.ds(s * AC + D + j * L, L)] = (
                bbuf[pl.ds(j * L, L)] - m * a)
        return 0
    lax.fori_loop(0, ZR, _seg, 0)
    pltpu.sync_copy(acstage, ac_sh.at[pl.ds(sid * ZR * AC, ZR * AC)])
    plsc.subcore_barrier()

    def _proc(c, rows):
        base = c * CH
        pltpu.sync_copy(feat_hbm.at[pl.ds(base * D, rows * D)],
                        buf.at[pl.ds(0, rows * D)])
        pltpu.sync_copy(seg_hbm.at[pl.ds(base, rows)],
                        segbuf.at[pl.ds(0, rows)])

        def _rows_apply(g, ks):
            avs = [acrow[pl.ds(j * L, L)] for j in range(2 * ND)]
            for k in ks:
                r = g * L + k
                for j in range(ND):
                    x = buf[pl.ds(r * D + j * L, L)]
                    buf[pl.ds(r * D + j * L, L)] = x * avs[j] + avs[ND + j]

        def _group(g, cur):
            segs = segbuf[pl.ds(g * L, L)]

            @pl.when(segs[L - 1] == cur)
            def _():
                _rows_apply(g, range(L))

            @pl.when(segs[L - 1] != cur)
            def _():
                for k in range(L):
                    prev = cur if k == 0 else segs[k - 1]

                    @pl.when(segs[k] != prev)
                    def _(s_k=segs[k]):
                        pltpu.sync_copy(ac_sh.at[pl.ds(s_k * AC, AC)], acrow)
                    _rows_apply(g, (k,))
            return segs[L - 1]

        lax.fori_loop(0, rows // L, _group, jnp.int32(-1))
        pltpu.sync_copy(buf.at[pl.ds(0, rows * D)],
                        out_hbm.at[pl.ds(base * D, rows * D)])

    def _trip(j, _):
        c = wid + j * NW

        @pl.when(c < T - 1)
        def _():
            _proc(c, CH)

        @pl.when(c == T - 1)
        def _():
            _proc(c, TAIL)
        return 0
    lax.fori_loop(0, JMAX, _trip, 0)


_SC_PARAMS = pltpu.CompilerParams(use_tc_tiling_on_sc=False,
                                  needs_layout_passes=False)

_stats_call = pl.kernel(
    _stats_body,
    out_type=jax.ShapeDtypeStruct((NC * B, W), jnp.float32),
    mesh=_MESH,
    compiler_params=_SC_PARAMS,
    scratch_types=[
        pltpu.VMEM((CH * D,), jnp.float32),
        pltpu.VMEM((CH + L,), jnp.int32),
        pltpu.VMEM((L, W), jnp.float32),
        pltpu.VMEM_SHARED((B, W), jnp.float32),
    ],
)

_apply_call = pl.kernel(
    _apply_body,
    out_type=jax.ShapeDtypeStruct((N * D,), jnp.float32),
    mesh=_MESH,
    compiler_params=_SC_PARAMS,
    scratch_types=[
        pltpu.VMEM((CH * D,), jnp.float32),
        pltpu.VMEM((CH + L,), jnp.int32),
        pltpu.VMEM((AC,), jnp.float32),
        pltpu.VMEM((ZR, W), jnp.float32),
        pltpu.VMEM((ZR, W), jnp.float32),
        pltpu.VMEM((ZR * AC,), jnp.float32),
        pltpu.VMEM((D,), jnp.float32),
        pltpu.VMEM((D,), jnp.float32),
        pltpu.VMEM((D,), jnp.float32),
        pltpu.VMEM_SHARED((B * AC,), jnp.float32),
    ],
)


@jax.jit
def kernel(feat, segment_ids, weight, bias, mean_scale):
    seg = segment_ids.astype(jnp.int32)
    feat_flat = feat.reshape(N * D)
    zero = jnp.zeros((ZR, W), jnp.float32)
    part = _stats_call(feat_flat, seg, zero)
    out_flat = _apply_call(feat_flat, seg, part, weight, bias, mean_scale)
    return out_flat.reshape(N, D)


# speedup vs baseline: 1.3368x; 1.3368x over previous
import functools

import jax
import jax.numpy as jnp
from jax import lax
from jax.experimental import pallas as pl
from jax.experimental.pallas import tpu as pltpu
from jax.experimental.pallas import tpu_sc as plsc

N = 100000
D = 128
B = 512
NC = 2
NS = 16
NW = NC * NS
L = 16
ND = D // L

W = 2 * D + L
ZR = B // NS
AC = 2 * D

CH1 = 448
T1 = -(-N // CH1)
F1 = T1 - 1
TAIL1 = N - F1 * CH1
J1 = -(-F1 // NW)
TW1 = F1 % NW

CH2 = 368
T2 = -(-N // CH2)
F2 = T2 - 1
TAIL2 = N - F2 * CH2
J2 = -(-F2 // NW)
TW2 = F2 % NW

_MESH = plsc.VectorSubcoreMesh(
    core_axis_name="c", subcore_axis_name="s", num_cores=NC, num_subcores=NS)
_SC_PARAMS = pltpu.CompilerParams(use_tc_tiling_on_sc=False,
                                  needs_layout_passes=False)


def _rsqrt_newton(v):
    i = plsc.bitcast(v, jnp.int32)
    i = jnp.int32(0x5F3759DF) - lax.shift_right_logical(i, 1)
    y = plsc.bitcast(i, jnp.float32)
    for _ in range(3):
        y = y * (1.5 - 0.5 * v * y * y)
    return y


def _mk_load(feat_hbm, seg_hbm, ch):
    def start(c, fb, sb, sem):
        base = c * ch
        pltpu.async_copy(feat_hbm.at[pl.ds(base * D, ch * D)], fb, sem)
        pltpu.async_copy(seg_hbm.at[pl.ds(base, ch)],
                         sb.at[pl.ds(0, ch)], sem)

    def wait(c, fb, sb, sem):
        base = c * ch
        pltpu.make_async_copy(feat_hbm.at[pl.ds(base * D, ch * D)],
                              fb, sem).wait()
        pltpu.make_async_copy(seg_hbm.at[pl.ds(base, ch)],
                              sb.at[pl.ds(0, ch)], sem).wait()
    return start, wait


def _stats_body(feat_hbm, seg_hbm, zero_hbm, part_hbm,
                fb0, fb1, sb0, sb1, stage, acc_sh, lsem0, lsem1):
    cid = lax.axis_index("c")
    sid = lax.axis_index("s")
    wid = sid * NC + cid
    fbs, sbs, lsems = (fb0, fb1), (sb0, sb1), (lsem0, lsem1)
    start_load, wait_load = _mk_load(feat_hbm, seg_hbm, CH1)

    start_load(wid, fb0, sb0, lsem0)

    pltpu.sync_copy(zero_hbm, acc_sh.at[pl.ds(sid * ZR, ZR)])
    pltpu.sync_copy(zero_hbm.at[pl.ds(0, L)], stage)
    plsc.subcore_barrier()

    lane = lax.iota(jnp.int32, L)
    row0 = jnp.zeros((L,), jnp.int32)

    def _flush(cur, cnt, accs, sqs):
        for j in range(ND):
            plsc.store_scatter(stage, [row0, lane + j * L], accs[j])
            plsc.store_scatter(stage, [row0, lane + D + j * L], sqs[j])
        plsc.store_scatter(stage, [row0, lane + 2 * D],
                           jnp.full((L,), cnt, jnp.float32))
        idxvec = jnp.full((L,), cur, jnp.int32)
        pltpu.sync_copy(stage, acc_sh.at[idxvec], add=True)

    def _compute(featbuf, segbuf, rows):
        zv = jnp.zeros((L,), jnp.float32)
        carry0 = ((segbuf[pl.ds(0, L)][0], jnp.float32(0.0))
                  + (zv,) * (2 * ND))

        def _group(g, carry):
            cur, cnt = carry[0], carry[1]
            accs = list(carry[2:2 + ND])
            sqs = list(carry[2 + ND:])
            segs = segbuf[pl.ds(g * L, L)]
            for k in range(L):
                s_k = segs[k]
                prev = cur if k == 0 else segs[k - 1]
                is_new = s_k != prev

                @pl.when(is_new)
                def _(cnt=cnt, accs=tuple(accs), sqs=tuple(sqs), prev=prev):
                    _flush(prev, cnt, accs, sqs)

                r = g * L + k
                xs = [featbuf[pl.ds(r * D + j * L, L)] for j in range(ND)]
                cnt = jnp.where(is_new, 1.0, cnt + 1.0)
                for j in range(ND):
                    accs[j] = jnp.where(is_new, xs[j], accs[j] + xs[j])
                    sqs[j] = jnp.where(is_new, xs[j] * xs[j],
                                       sqs[j] + xs[j] * xs[j])
            return (segs[L - 1], cnt) + tuple(accs) + tuple(sqs)

        end = lax.fori_loop(0, rows // L, _group, carry0)
        _flush(end[0], end[1], end[2:2 + ND], end[2 + ND:])

    def _pair(p, _):
        c0 = wid + 2 * p * NW
        c1 = c0 + NW

        @pl.when(c0 < F1)
        def _():
            @pl.when(c1 < F1)
            def _():
                start_load(c1, fb1, sb1, lsem1)
            wait_load(c0, fb0, sb0, lsem0)
            _compute(fb0, sb0, CH1)

        @pl.when(c1 < F1)
        def _():
            @pl.when(c1 + NW < F1)
            def _():
                start_load(c1 + NW, fb0, sb0, lsem0)
            wait_load(c1, fb1, sb1, lsem1)
            _compute(fb1, sb1, CH1)
        return 0
    lax.fori_loop(0, (J1 + 1) // 2, _pair, 0)

    @pl.when(wid == TW1)
    def _():
        base = F1 * CH1
        pltpu.sync_copy(feat_hbm.at[pl.ds(base * D, TAIL1 * D)],
                        fb0.at[pl.ds(0, TAIL1 * D)])
        pltpu.sync_copy(seg_hbm.at[pl.ds(base, TAIL1)],
                        sb0.at[pl.ds(0, TAIL1)])
        _compute(fb0, sb0, TAIL1)

    plsc.subcore_barrier()
    pltpu.sync_copy(acc_sh.at[pl.ds(sid * ZR, ZR)],
                    part_hbm.at[pl.ds(cid * B + sid * ZR, ZR)])


def _apply_body(feat_hbm, seg_hbm, part_hbm, w_hbm, b_hbm, ms_hbm, out_hbm,
                fb0, fb1, sb0, sb1, acrow, p0buf, p1buf, acstage,
                wbuf, bbuf, msbuf, ac_sh, lsem0, lsem1, ssem0, ssem1):
    cid = lax.axis_index("c")
    sid = lax.axis_index("s")
    wid = sid * NC + cid
    fbs, sbs = (fb0, fb1), (sb0, sb1)
    lsems, ssems = (lsem0, lsem1), (ssem0, ssem1)
    start_load, wait_load = _mk_load(feat_hbm, seg_hbm, CH2)

    def _start_store(c, fb, sem):
        pltpu.async_copy(fb, out_hbm.at[pl.ds(c * CH2 * D, CH2 * D)], sem)

    def _wait_store(c, fb, sem):
        pltpu.make_async_copy(fb, out_hbm.at[pl.ds(c * CH2 * D, CH2 * D)],
                              sem).wait()

    start_load(wid, fb0, sb0, lsem0)

    pltpu.sync_copy(w_hbm, wbuf)
    pltpu.sync_copy(b_hbm, bbuf)
    pltpu.sync_copy(ms_hbm, msbuf)
    pltpu.sync_copy(part_hbm.at[pl.ds(sid * ZR, ZR)], p0buf)
    pltpu.sync_copy(part_hbm.at[pl.ds(B + sid * ZR, ZR)], p1buf)

    def _seg(s, _):
        cntv = p0buf[s, pl.ds(2 * D, L)] + p1buf[s, pl.ds(2 * D, L)]
        cnt = jnp.maximum(cntv, 1.0)
        for j in range(ND):
            sm = p0buf[s, pl.ds(j * L, L)] + p1buf[s, pl.ds(j * L, L)]
            sq = (p0buf[s, pl.ds(D + j * L, L)]
                  + p1buf[s, pl.ds(D + j * L, L)])
            mean = sm / cnt
            m = mean * msbuf[pl.ds(j * L, L)]
            var = sq / cnt - m * (2.0 * mean - m)
            y = _rsqrt_newton(var + 1e-6)
            a = wbuf[pl.ds(j * L, L)] * y
            acstage[pl.ds(s * AC + j * L, L)] = a
            acstage[pl.ds(s * AC + D + j * L, L)] = (
                bbuf[pl.ds(j * L, L)] - m * a)
        return 0
    lax.fori_loop(0, ZR, _seg, 0)
    pltpu.sync_copy(acstage, ac_sh.at[pl.ds(sid * ZR * AC, ZR * AC)])
    plsc.subcore_barrier()

    def _compute(buf, segbuf, rows):
        def _rows_apply(g, ks):
            avs = [acrow[pl.ds(j * L, L)] for j in range(2 * ND)]
            for k in ks:
                r = g * L + k
                for j in range(ND):
                    x = buf[pl.ds(r * D + j * L, L)]
                    buf[pl.ds(r * D + j * L, L)] = x * avs[j] + avs[ND + j]

        def _group(g, cur):
            segs = segbuf[pl.ds(g * L, L)]

            @pl.when(segs[L - 1] == cur)
            def _():
                _rows_apply(g, range(L))

            @pl.when(segs[L - 1] != cur)
            def _():
                for k in range(L):
                    prev = cur if k == 0 else segs[k - 1]

                    @pl.when(segs[k] != prev)
                    def _(s_k=segs[k]):
                        pltpu.sync_copy(ac_sh.at[pl.ds(s_k * AC, AC)], acrow)
                    _rows_apply(g, (k,))
            return segs[L - 1]

        lax.fori_loop(0, rows // L, _group, jnp.int32(-1))

    def _pair(p, _):
        c0 = wid + 2 * p * NW
        c1 = c0 + NW

        @pl.when(c0 < F2)
        def _():
            @pl.when(c1 < F2)
            def _():
                @pl.when(c1 >= 2 * NW)
                def _():
                    _wait_store(c1 - 2 * NW, fb1, ssem1)
                start_load(c1, fb1, sb1, lsem1)
            wait_load(c0, fb0, sb0, lsem0)
            _compute(fb0, sb0, CH2)
            _start_store(c0, fb0, ssem0)

        @pl.when(c1 < F2)
        def _():
            @pl.when(c1 + NW < F2)
            def _():
                _wait_store(c0, fb0, ssem0)
                start_load(c1 + NW, fb0, sb0, lsem0)
            wait_load(c1, fb1, sb1, lsem1)
            _compute(fb1, sb1, CH2)
            _start_store(c1, fb1, ssem1)
        return 0
    lax.fori_loop(0, (J2 + 1) // 2, _pair, 0)

    for j in range(J2):
        c = wid + j * NW

        @pl.when((c < F2) & (c + 2 * NW >= F2))
        def _(c=c, b=j % 2):
            _wait_store(c, fbs[b], ssems[b])

    @pl.when(wid == TW2)
    def _():
        base = F2 * CH2
        pltpu.sync_copy(feat_hbm.at[pl.ds(base * D, TAIL2 * D)],
                        fb0.at[pl.ds(0, TAIL2 * D)])
        pltpu.sync_copy(seg_hbm.at[pl.ds(base, TAIL2)],
                        sb0.at[pl.ds(0, TAIL2)])
        _compute(fb0, sb0, TAIL2)
        pltpu.sync_copy(fb0.at[pl.ds(0, TAIL2 * D)],
                        out_hbm.at[pl.ds(base * D, TAIL2 * D)])


_stats_call = pl.kernel(
    _stats_body,
    out_type=jax.ShapeDtypeStruct((NC * B, W), jnp.float32),
    mesh=_MESH,
    compiler_params=_SC_PARAMS,
    scratch_types=[
        pltpu.VMEM((CH1 * D,), jnp.float32),
        pltpu.VMEM((CH1 * D,), jnp.float32),
        pltpu.VMEM((CH1 + L,), jnp.int32),
        pltpu.VMEM((CH1 + L,), jnp.int32),
        pltpu.VMEM((L, W), jnp.float32),
        pltpu.VMEM_SHARED((B, W), jnp.float32),
        pltpu.SemaphoreType.DMA,
        pltpu.SemaphoreType.DMA,
    ],
)

_apply_call = pl.kernel(
    _apply_body,
    out_type=jax.ShapeDtypeStruct((N * D,), jnp.float32),
    mesh=_MESH,
    compiler_params=_SC_PARAMS,
    scratch_types=[
        pltpu.VMEM((CH2 * D,), jnp.float32),
        pltpu.VMEM((CH2 * D,), jnp.float32),
        pltpu.VMEM((CH2 + L,), jnp.int32),
        pltpu.VMEM((CH2 + L,), jnp.int32),
        pltpu.VMEM((AC,), jnp.float32),
        pltpu.VMEM((ZR, W), jnp.float32),
        pltpu.VMEM((ZR, W), jnp.float32),
        pltpu.VMEM((ZR * AC,), jnp.float32),
        pltpu.VMEM((D,), jnp.float32),
        pltpu.VMEM((D,), jnp.float32),
        pltpu.VMEM((D,), jnp.float32),
        pltpu.VMEM_SHARED((B * AC,), jnp.float32),
        pltpu.SemaphoreType.DMA,
        pltpu.SemaphoreType.DMA,
        pltpu.SemaphoreType.DMA,
        pltpu.SemaphoreType.DMA,
    ],
)


@jax.jit
def kernel(feat, segment_ids, weight, bias, mean_scale):
    seg = segment_ids.astype(jnp.int32)
    feat_flat = feat.reshape(N * D)
    zero = jnp.zeros((ZR, W), jnp.float32)
    part = _stats_call(feat_flat, seg, zero)
    out_flat = _apply_call(feat_flat, seg, part, weight, bias, mean_scale)
    return out_flat.reshape(N, D)


# speedup vs baseline: 1.9315x; 1.4449x over previous
import functools

import jax
import jax.numpy as jnp
from jax import lax
from jax.experimental import pallas as pl
from jax.experimental.pallas import tpu as pltpu
from jax.experimental.pallas import tpu_sc as plsc

N = 100000
D = 128
B = 512
NC = 2
NS = 16
NW = NC * NS
L = 16
ND = D // L

W = 2 * D + L
ZR = B // NS
AC = 2 * D

CH1 = 448
T1 = -(-N // CH1)
F1 = T1 - 1
TAIL1 = N - F1 * CH1
J1 = -(-F1 // NW)
TW1 = F1 % NW

CH2 = 368
T2 = -(-N // CH2)
F2 = T2 - 1
TAIL2 = N - F2 * CH2
J2 = -(-F2 // NW)
TW2 = F2 % NW

_MESH = plsc.VectorSubcoreMesh(
    core_axis_name="c", subcore_axis_name="s", num_cores=NC, num_subcores=NS)
_SC_PARAMS = pltpu.CompilerParams(use_tc_tiling_on_sc=False,
                                  needs_layout_passes=False)


def _rsqrt_newton(v):
    i = plsc.bitcast(v, jnp.int32)
    i = jnp.int32(0x5F3759DF) - lax.shift_right_logical(i, 1)
    y = plsc.bitcast(i, jnp.float32)
    for _ in range(3):
        y = y * (1.5 - 0.5 * v * y * y)
    return y


def _mk_load(feat_hbm, seg_hbm, ch):
    def start(c, fb, sb, sem):
        base = c * ch
        pltpu.async_copy(feat_hbm.at[pl.ds(base * D, ch * D)], fb, sem)
        pltpu.async_copy(seg_hbm.at[pl.ds(base, ch)],
                         sb.at[pl.ds(0, ch)], sem)

    def wait(c, fb, sb, sem):
        base = c * ch
        pltpu.make_async_copy(feat_hbm.at[pl.ds(base * D, ch * D)],
                              fb, sem).wait()
        pltpu.make_async_copy(seg_hbm.at[pl.ds(base, ch)],
                              sb.at[pl.ds(0, ch)], sem).wait()
    return start, wait


def _stats_body(feat_hbm, seg_hbm, zero_hbm, part_hbm,
                fb0, fb1, sb0, sb1, stage, accbuf, acc_sh, lsem0, lsem1):
    cid = lax.axis_index("c")
    sid = lax.axis_index("s")
    wid = sid * NC + cid
    fbs, sbs, lsems = (fb0, fb1), (sb0, sb1), (lsem0, lsem1)
    start_load, wait_load = _mk_load(feat_hbm, seg_hbm, CH1)

    start_load(wid, fb0, sb0, lsem0)

    pltpu.sync_copy(zero_hbm, acc_sh.at[pl.ds(sid * ZR, ZR)])
    pltpu.sync_copy(zero_hbm.at[pl.ds(0, L)], stage)
    plsc.subcore_barrier()

    lane = lax.iota(jnp.int32, L)
    row0 = jnp.zeros((L,), jnp.int32)
    zv = jnp.zeros((L,), jnp.float32)
    one16 = jnp.full((L,), 1.0, jnp.float32)
    grp16 = jnp.full((L,), float(L), jnp.float32)

    def _zero_accbuf():
        for t in range(W // L):
            accbuf[pl.ds(t * L, L)] = zv
    _zero_accbuf()

    def _flush(cur):
        for t in range(W // L):
            plsc.store_scatter(stage, [row0, lane + t * L],
                               accbuf[pl.ds(t * L, L)])
        idxvec = jnp.full((L,), cur, jnp.int32)
        pltpu.sync_copy(stage, acc_sh.at[idxvec], add=True)
        _zero_accbuf()

    def _compute(featbuf, segbuf, rows):
        def _acc_row(r):
            xs = [featbuf[pl.ds(r * D + j * L, L)] for j in range(ND)]
            for j in range(ND):
                plsc.addupdate(accbuf.at[pl.ds(j * L, L)], xs[j])
                plsc.addupdate(accbuf.at[pl.ds(D + j * L, L)],
                               xs[j] * xs[j])
            plsc.addupdate(accbuf.at[pl.ds(2 * D, L)], one16)

        def _group(g, cur):
            segs = segbuf[pl.ds(g * L, L)]
            same = (segs[0] == cur) & (segs[L - 1] == cur)

            @pl.when(same)
            def _():
                gs = [zv] * ND
                gq = [zv] * ND
                for k in range(L):
                    r = g * L + k
                    xs = [featbuf[pl.ds(r * D + j * L, L)]
                          for j in range(ND)]
                    for j in range(ND):
                        gs[j] = gs[j] + xs[j]
                        gq[j] = gq[j] + xs[j] * xs[j]
                for j in range(ND):
                    plsc.addupdate(accbuf.at[pl.ds(j * L, L)], gs[j])
                    plsc.addupdate(accbuf.at[pl.ds(D + j * L, L)], gq[j])
                plsc.addupdate(accbuf.at[pl.ds(2 * D, L)], grp16)

            @pl.when(jnp.logical_not(same))
            def _():
                for k in range(L):
                    prev = cur if k == 0 else segs[k - 1]

                    @pl.when(segs[k] != prev)
                    def _(prev=prev):
                        _flush(prev)
                    _acc_row(g * L + k)
            return segs[L - 1]

        cur0 = segbuf[pl.ds(0, L)][0]
        end = lax.fori_loop(0, rows // L, _group, cur0)
        _flush(end)

    def _pair(p, _):
        c0 = wid + 2 * p * NW
        c1 = c0 + NW

        @pl.when(c0 < F1)
        def _():
            @pl.when(c1 < F1)
            def _():
                start_load(c1, fb1, sb1, lsem1)
            wait_load(c0, fb0, sb0, lsem0)
            _compute(fb0, sb0, CH1)

        @pl.when(c1 < F1)
        def _():
            @pl.when(c1 + NW < F1)
            def _():
                start_load(c1 + NW, fb0, sb0, lsem0)
            wait_load(c1, fb1, sb1, lsem1)
            _compute(fb1, sb1, CH1)
        return 0
    lax.fori_loop(0, (J1 + 1) // 2, _pair, 0)

    @pl.when(wid == TW1)
    def _():
        base = F1 * CH1
        pltpu.sync_copy(feat_hbm.at[pl.ds(base * D, TAIL1 * D)],
                        fb0.at[pl.ds(0, TAIL1 * D)])
        pltpu.sync_copy(seg_hbm.at[pl.ds(base, TAIL1)],
                        sb0.at[pl.ds(0, TAIL1)])
        _compute(fb0, sb0, TAIL1)

    plsc.subcore_barrier()
    pltpu.sync_copy(acc_sh.at[pl.ds(sid * ZR, ZR)],
                    part_hbm.at[pl.ds(cid * B + sid * ZR, ZR)])


def _apply_body(feat_hbm, seg_hbm, part_hbm, w_hbm, b_hbm, ms_hbm, out_hbm,
                fb0, fb1, sb0, sb1, acrow, p0buf, p1buf, acstage,
                wbuf, bbuf, msbuf, ac_sh, lsem0, lsem1, ssem0, ssem1):
    cid = lax.axis_index("c")
    sid = lax.axis_index("s")
    wid = sid * NC + cid
    fbs, sbs = (fb0, fb1), (sb0, sb1)
    lsems, ssems = (lsem0, lsem1), (ssem0, ssem1)
    start_load, wait_load = _mk_load(feat_hbm, seg_hbm, CH2)

    def _start_store(c, fb, sem):
        pltpu.async_copy(fb, out_hbm.at[pl.ds(c * CH2 * D, CH2 * D)], sem)

    def _wait_store(c, fb, sem):
        pltpu.make_async_copy(fb, out_hbm.at[pl.ds(c * CH2 * D, CH2 * D)],
                              sem).wait()

    start_load(wid, fb0, sb0, lsem0)

    pltpu.sync_copy(w_hbm, wbuf)
    pltpu.sync_copy(b_hbm, bbuf)
    pltpu.sync_copy(ms_hbm, msbuf)
    pltpu.sync_copy(part_hbm.at[pl.ds(sid * ZR, ZR)], p0buf)
    pltpu.sync_copy(part_hbm.at[pl.ds(B + sid * ZR, ZR)], p1buf)

    def _seg(s, _):
        cntv = p0buf[s, pl.ds(2 * D, L)] + p1buf[s, pl.ds(2 * D, L)]
        cnt = jnp.maximum(cntv, 1.0)
        for j in range(ND):
            sm = p0buf[s, pl.ds(j * L, L)] + p1buf[s, pl.ds(j * L, L)]
            sq = (p0buf[s, pl.ds(D + j * L, L)]
                  + p1buf[s, pl.ds(D + j * L, L)])
            mean = sm / cnt
            m = mean * msbuf[pl.ds(j * L, L)]
            var = sq / cnt - m * (2.0 * mean - m)
            y = _rsqrt_newton(var + 1e-6)
            a = wbuf[pl.ds(j * L, L)] * y
            acstage[pl.ds(s * AC + j * L, L)] = a
            acstage[pl.ds(s * AC + D + j * L, L)] = (
                bbuf[pl.ds(j * L, L)] - m * a)
        return 0
    lax.fori_loop(0, ZR, _seg, 0)
    pltpu.sync_copy(acstage, ac_sh.at[pl.ds(sid * ZR * AC, ZR * AC)])
    plsc.subcore_barrier()

    def _compute(buf, segbuf, rows):
        def _rows_apply(g, ks):
            avs = [acrow[pl.ds(j * L, L)] for j in range(2 * ND)]
            for k in ks:
                r = g * L + k
                xs = [buf[pl.ds(r * D + j * L, L)] for j in range(ND)]
                ys = [xs[j] * avs[j] + avs[ND + j] for j in range(ND)]
                for j in range(ND):
                    buf[pl.ds(r * D + j * L, L)] = ys[j]

        def _group(g, cur):
            segs = segbuf[pl.ds(g * L, L)]

            @pl.when(segs[L - 1] == cur)
            def _():
                _rows_apply(g, range(L))

            @pl.when(segs[L - 1] != cur)
            def _():
                for k in range(L):
                    prev = cur if k == 0 else segs[k - 1]

                    @pl.when(segs[k] != prev)
                    def _(s_k=segs[k]):
                        pltpu.sync_copy(ac_sh.at[pl.ds(s_k * AC, AC)], acrow)
                    _rows_apply(g, (k,))
            return segs[L - 1]

        lax.fori_loop(0, rows // L, _group, jnp.int32(-1))

    def _pair(p, _):
        c0 = wid + 2 * p * NW
        c1 = c0 + NW

        @pl.when(c0 < F2)
        def _():
            @pl.when(c1 < F2)
            def _():
                @pl.when(c1 >= 2 * NW)
                def _():
                    _wait_store(c1 - 2 * NW, fb1, ssem1)
                start_load(c1, fb1, sb1, lsem1)
            wait_load(c0, fb0, sb0, lsem0)
            _compute(fb0, sb0, CH2)
            _start_store(c0, fb0, ssem0)

        @pl.when(c1 < F2)
        def _():
            @pl.when(c1 + NW < F2)
            def _():
                _wait_store(c0, fb0, ssem0)
                start_load(c1 + NW, fb0, sb0, lsem0)
            wait_load(c1, fb1, sb1, lsem1)
            _compute(fb1, sb1, CH2)
            _start_store(c1, fb1, ssem1)
        return 0
    lax.fori_loop(0, (J2 + 1) // 2, _pair, 0)

    for j in range(J2):
        c = wid + j * NW

        @pl.when((c < F2) & (c + 2 * NW >= F2))
        def _(c=c, b=j % 2):
            _wait_store(c, fbs[b], ssems[b])

    @pl.when(wid == TW2)
    def _():
        base = F2 * CH2
        pltpu.sync_copy(feat_hbm.at[pl.ds(base * D, TAIL2 * D)],
                        fb0.at[pl.ds(0, TAIL2 * D)])
        pltpu.sync_copy(seg_hbm.at[pl.ds(base, TAIL2)],
                        sb0.at[pl.ds(0, TAIL2)])
        _compute(fb0, sb0, TAIL2)
        pltpu.sync_copy(fb0.at[pl.ds(0, TAIL2 * D)],
                        out_hbm.at[pl.ds(base * D, TAIL2 * D)])


_stats_call = pl.kernel(
    _stats_body,
    out_type=jax.ShapeDtypeStruct((NC * B, W), jnp.float32),
    mesh=_MESH,
    compiler_params=_SC_PARAMS,
    scratch_types=[
        pltpu.VMEM((CH1 * D,), jnp.float32),
        pltpu.VMEM((CH1 * D,), jnp.float32),
        pltpu.VMEM((CH1 + L,), jnp.int32),
        pltpu.VMEM((CH1 + L,), jnp.int32),
        pltpu.VMEM((L, W), jnp.float32),
        pltpu.VMEM((W,), jnp.float32),
        pltpu.VMEM_SHARED((B, W), jnp.float32),
        pltpu.SemaphoreType.DMA,
        pltpu.SemaphoreType.DMA,
    ],
)

_apply_call = pl.kernel(
    _apply_body,
    out_type=jax.ShapeDtypeStruct((N * D,), jnp.float32),
    mesh=_MESH,
    compiler_params=_SC_PARAMS,
    scratch_types=[
        pltpu.VMEM((CH2 * D,), jnp.float32),
        pltpu.VMEM((CH2 * D,), jnp.float32),
        pltpu.VMEM((CH2 + L,), jnp.int32),
        pltpu.VMEM((CH2 + L,), jnp.int32),
        pltpu.VMEM((AC,), jnp.float32),
        pltpu.VMEM((ZR, W), jnp.float32),
        pltpu.VMEM((ZR, W), jnp.float32),
        pltpu.VMEM((ZR * AC,), jnp.float32),
        pltpu.VMEM((D,), jnp.float32),
        pltpu.VMEM((D,), jnp.float32),
        pltpu.VMEM((D,), jnp.float32),
        pltpu.VMEM_SHARED((B * AC,), jnp.float32),
        pltpu.SemaphoreType.DMA,
        pltpu.SemaphoreType.DMA,
        pltpu.SemaphoreType.DMA,
        pltpu.SemaphoreType.DMA,
    ],
)


@jax.jit
def kernel(feat, segment_ids, weight, bias, mean_scale):
    seg = segment_ids.astype(jnp.int32)
    feat_flat = feat.reshape(N * D)
    zero = jnp.zeros((ZR, W), jnp.float32)
    part = _stats_call(feat_flat, seg, zero)
    out_flat = _apply_call(feat_flat, seg, part, weight, bias, mean_scale)
    return out_flat.reshape(N, D)
